# baseline (speedup 1.0000x reference)
import functools

import jax
import jax.numpy as jnp
from jax import lax
from jax.experimental import pallas as pl
from jax.experimental.pallas import tpu as pltpu
from jax.experimental.pallas import tpu_sc as plsc

B = 4
N_NODES = 2500
N_TOTAL = B * N_NODES
N_PAD = 10240
E = 320000
D = 128
H = 64
NB = N_PAD // 512
NS = 16
EPS = 1e-16


def _dot(a, b):
    return jnp.dot(a, b, preferred_element_type=jnp.float32)


def _halves(c, a):
    return jnp.where(c == 0, a[:, :H], a[:, H:])


def _pre_body_common(c, g, wl_ref, bl_ref, wi_ref, ab_ref, wj_ref,
                     ajx_ref, init_ref):
    xl = _dot(g, wl_ref[...]) + bl_ref[0:1, :]
    aj = _dot(xl, wj_ref[...])
    ai = _dot(xl, wi_ref[...]) + ab_ref[0:1, :]
    xlh = _halves(c, xl)
    ajh = _halves(c, aj)
    aih = _halves(c, ai)
    s = aih + ajh
    le = jnp.exp(jnp.maximum(s, 0.2 * s))
    ajx_ref[...] = jnp.where(c == 2, ai, jnp.concatenate([ajh, xlh], axis=1))
    init_ref[...] = jnp.concatenate([le, xlh * le], axis=1)


def _pre0_kernel(g_ref, wl_ref, bl_ref, wi_ref, ab_ref, wj_ref,
                 ajx_ref, init_ref):
    c = pl.program_id(0)
    _pre_body_common(c, g_ref[...], wl_ref, bl_ref, wi_ref, ab_ref, wj_ref,
                     ajx_ref, init_ref)


def _pre1_kernel(e0_ref, e1_ref, wl_ref, bl_ref, wi_ref, ab_ref, wj_ref,
                 ajx_ref, init_ref):
    c = pl.program_id(0)
    e0 = e0_ref[...]
    e1 = e1_ref[...]
    g = jnp.concatenate([e0[:, H:] / (e0[:, :H] + EPS),
                         e1[:, H:] / (e1[:, :H] + EPS)], axis=1)
    _pre_body_common(c, g, wl_ref, bl_ref, wi_ref, ab_ref, wj_ref,
                     ajx_ref, init_ref)


def _pre_out_shapes():
    return [
        jax.ShapeDtypeStruct((3 * N_PAD, D), jnp.float32),
        jax.ShapeDtypeStruct((3 * N_PAD, D), jnp.float32),
    ]


_W_SPEC = pl.BlockSpec((D, D), lambda c, i: (0, 0))
_B_SPEC = pl.BlockSpec((8, D), lambda c, i: (0, 0))
_PRE_OUT_SPECS = [
    pl.BlockSpec((512, D), lambda c, i: (c * NB + i, 0)),
    pl.BlockSpec((512, D), lambda c, i: (c * NB + i, 0)),
]


def _pre0(g, wlT, bl2, wiT, ab2, wjT):
    return pl.pallas_call(
        _pre0_kernel,
        grid=(3, NB),
        in_specs=[pl.BlockSpec((512, D), lambda c, i: (i, 0)),
                  _W_SPEC, _B_SPEC, _W_SPEC, _B_SPEC, _W_SPEC],
        out_specs=_PRE_OUT_SPECS,
        out_shape=_pre_out_shapes(),
    )(g, wlT, bl2, wiT, ab2, wjT)


def _pre1(eout, wlT, bl2, wiT, ab2, wjT):
    return pl.pallas_call(
        _pre1_kernel,
        grid=(3, NB),
        in_specs=[pl.BlockSpec((512, D), lambda c, i: (i, 0)),
                  pl.BlockSpec((512, D), lambda c, i: (NB + i, 0)),
                  _W_SPEC, _B_SPEC, _W_SPEC, _B_SPEC, _W_SPEC],
        out_specs=_PRE_OUT_SPECS,
        out_shape=_pre_out_shapes(),
    )(eout, eout, wlT, bl2, wiT, ab2, wjT)


EP = 322560
EPW = EP // NS
CHUNK = 48
SUPER = 2016
NSUPER = EPW // SUPER
CPS = SUPER // CHUNK
ROWS_PW = N_PAD // NS
PAD_ROW = 10200


def _edge_body(tt_hbm, init_hbm, src_hbm, dst_hbm, out_hbm,
               src_all, dst_all,
               gi0, dsti0, gv0, out0,
               gi1, dsti1, gv1, out1,
               acc, gsem0, gsem1):
    c = lax.axis_index("c")
    sid = lax.axis_index("s")
    coff = (c * N_PAD).astype(jnp.int32)
    aoff = jnp.int32(2 * N_PAD)
    choff = c * H
    ebase = sid * EPW
    rbase = sid * ROWS_PW
    slots = ((gi0, dsti0, gv0, out0, gsem0),
             (gi1, dsti1, gv1, out1, gsem1))

    pltpu.sync_copy(init_hbm.at[pl.ds(c * N_PAD + rbase, ROWS_PW)],
                    acc.at[pl.ds(rbase, ROWS_PW)])
    plsc.subcore_barrier()

    def prep(slot, t):
        gi, dsti, gv, _, gsem = slot
        tb = t * CHUNK
        for i in range(CHUNK // 16):
            sv = src_all[pl.ds(tb + i * 16, 16)]
            dv = dst_all[pl.ds(tb + i * 16, 16)]
            gi[pl.ds(i * 16, 16)] = sv + coff
            gi[pl.ds(CHUNK + i * 16, 16)] = dv + aoff
            dsti[pl.ds(i * 16, 16)] = dv
        pltpu.async_copy(tt_hbm.at[gi], gv, gsem)

    def consume(slot):
        gi, dsti, gv, out_v, gsem = slot
        pltpu.make_async_copy(tt_hbm.at[gi], gv, gsem).wait()

        @plsc.parallel_loop(0, CHUNK, unroll=4)
        def _(e):
            for k in range(H // 16):
                a = (gv[CHUNK + e, pl.ds(choff + k * 16, 16)]
                     + gv[e, pl.ds(k * 16, 16)])
                ex = jnp.exp(jnp.maximum(a, 0.2 * a))
                out_v[e, pl.ds(k * 16, 16)] = ex
                out_v[e, pl.ds(H + k * 16, 16)] = (
                    gv[e, pl.ds(H + k * 16, 16)] * ex)

        pltpu.sync_copy(out_v, acc.at[dsti], add=True)

    def super_body(u, _):
        sbase = ebase + u * SUPER
        pltpu.sync_copy(src_hbm.at[pl.ds(sbase, SUPER)], src_all)
        pltpu.sync_copy(dst_hbm.at[pl.ds(sbase, SUPER)], dst_all)
        prep(slots[0], 0)
        prep(slots[1], 1)

        def round_body(tt, _):
            for s in range(2):
                consume(slots[s])
                prep(slots[s], 2 * tt + 2 + s)
            return 0

        lax.fori_loop(0, CPS // 2 - 1, round_body, 0)
        consume(slots[0])
        consume(slots[1])
        return 0

    lax.fori_loop(0, NSUPER, super_body, 0)
    plsc.subcore_barrier()
    pltpu.sync_copy(acc.at[pl.ds(rbase, ROWS_PW)],
                    out_hbm.at[pl.ds(c * N_PAD + rbase, ROWS_PW)])


def _edge_pass(tt, init, src, dst):
    mesh = plsc.VectorSubcoreMesh(core_axis_name="c", subcore_axis_name="s",
                                  num_cores=2, num_subcores=NS)
    f = pl.kernel(
        _edge_body,
        out_type=jax.ShapeDtypeStruct((2 * N_PAD, D), jnp.float32),
        mesh=mesh,
        scratch_types=(
            [pltpu.VMEM((SUPER,), jnp.int32)] * 2
            + [pltpu.VMEM((2 * CHUNK,), jnp.int32),
               pltpu.VMEM((CHUNK,), jnp.int32),
               pltpu.VMEM((2 * CHUNK, D), jnp.float32),
               pltpu.VMEM((CHUNK, D), jnp.float32)] * 2
            + [pltpu.VMEM_SHARED((N_PAD, D), jnp.float32),
               pltpu.SemaphoreType.DMA,
               pltpu.SemaphoreType.DMA]
        ),
    )
    return f(tt, init, src, dst)


def _final_kernel(e0_ref, e1_ref, it_ref, st_ref, winT_ref,
                  wzsT_ref, wziT_ref, wrsT_ref, wriT_ref, whsT_ref, whiT_ref,
                  wp_ref, ws_ref, bs_ref,
                  pcol_ref, scol_ref, hout_ref):
    e0 = e0_ref[...]
    e1 = e1_ref[...]
    g1 = jnp.concatenate([e0[:, H:] / (e0[:, :H] + EPS),
                          e1[:, H:] / (e1[:, :H] + EPS)], axis=1)

    it = it_ref[...]
    m = jnp.zeros((8, D), jnp.float32)
    for l in range(20):
        m = m + it[l * 8:(l + 1) * 8, :]
    m = m * (1.0 / 20.0)
    inp = _dot(m, winT_ref[...])
    st = st_ref[...]
    z = jax.nn.sigmoid(_dot(st, wzsT_ref[...]) + _dot(inp, wziT_ref[...]))
    r = jax.nn.sigmoid(_dot(st, wrsT_ref[...]) + _dot(inp, wriT_ref[...]))
    hc = jnp.tanh(_dot(r * st, whsT_ref[...]) + _dot(inp, whiT_ref[...]))
    hh = (1.0 - z) * st + z * hc
    hout_ref[...] = hh

    n = lax.broadcasted_iota(jnp.int32, (N_PAD, 1), 0)
    b_id = n // N_NODES
    hh_exp = jnp.where(b_id == 0, hh[0:1, :],
             jnp.where(b_id == 1, hh[1:2, :],
             jnp.where(b_id == 2, hh[2:3, :], hh[3:4, :])))
    lp = jnp.sum(g1 * (hh_exp * wp_ref[0:1, :]), axis=1, keepdims=True)
    ls = jnp.sum(g1 * (hh_exp * ws_ref[0:1, :]), axis=1, keepdims=True)
    scol_ref[...] = jax.nn.sigmoid(ls + bs_ref[0:1, 0:1])

    valid = (n % N_NODES != 0) & (n < N_TOTAL)
    lpm = jnp.where(valid, lp, -1e30)
    mx = jnp.zeros((N_PAD, 1), jnp.float32)
    for b in range(B):
        inb = b_id == b
        mb = jnp.max(jnp.where(inb, lpm, -1e30))
        mx = jnp.where(inb, mb, mx)
    ex = jnp.where(valid, jnp.exp(lp - mx), 0.0)
    sm = jnp.zeros((N_PAD, 1), jnp.float32)
    for b in range(B):
        inb = b_id == b
        sb = jnp.sum(jnp.where(inb, ex, 0.0))
        sm = jnp.where(inb, sb, sm)
    pcol_ref[...] = ex / sm


def _final(eout1, it, st, winT, wzsT, wziT, wrsT, wriT, whsT, whiT,
           wp2, ws2, bs2):
    full = lambda r: pl.BlockSpec((r, D), lambda i: (0, 0))
    return pl.pallas_call(
        _final_kernel,
        grid=(1,),
        in_specs=[pl.BlockSpec((N_PAD, D), lambda i: (0, 0)),
                  pl.BlockSpec((N_PAD, D), lambda i: (1, 0)),
                  full(160), full(8), full(D),
                  full(D), full(D), full(D), full(D), full(D), full(D),
                  full(8), full(8), full(8)],
        out_specs=[pl.BlockSpec((N_PAD, 1), lambda i: (0, 0)),
                   pl.BlockSpec((N_PAD, 1), lambda i: (0, 0)),
                   pl.BlockSpec((8, D), lambda i: (0, 0))],
        out_shape=[jax.ShapeDtypeStruct((N_PAD, 1), jnp.float32),
                   jax.ShapeDtypeStruct((N_PAD, 1), jnp.float32),
                   jax.ShapeDtypeStruct((8, D), jnp.float32)],
    )(eout1, eout1, it, st, winT, wzsT, wziT, wrsT, wriT, whsT, whiT,
      wp2, ws2, bs2)


def kernel(x, edge_index_0, edge_index_1, state_, input_, W_in, W_z, W_r,
           W_h, gat0_lin_w, gat0_lin_b, gat0_attn_w, gat0_attn_b,
           gat1_lin_w, gat1_lin_b, gat1_attn_w, gat1_attn_b,
           lin_prob_w, lin_prob_b, lin_sisr_w, lin_sisr_b):
    f32 = jnp.float32
    xp = jnp.pad(x, ((0, N_PAD - N_TOTAL), (0, 0)))

    def pre_args(lin_w, lin_b, attn_w, attn_b):
        return (lin_w.T.astype(f32),
                jnp.pad(lin_b[None, :], ((0, 7), (0, 0))),
                attn_w[:, :D].T.astype(f32),
                jnp.pad(attn_b[None, :], ((0, 7), (0, 0))),
                attn_w[:, D:].T.astype(f32))

    def pad_e(v):
        return jnp.pad(v, (0, EP - E), constant_values=PAD_ROW)

    tt0, init0 = _pre0(xp, *pre_args(gat0_lin_w, gat0_lin_b,
                                     gat0_attn_w, gat0_attn_b))
    e0out = _edge_pass(tt0, init0,
                       pad_e(edge_index_0[0]), pad_e(edge_index_0[1]))
    tt1, init1 = _pre1(e0out, *pre_args(gat1_lin_w, gat1_lin_b,
                                        gat1_attn_w, gat1_attn_b))
    e1out = _edge_pass(tt1, init1,
                       pad_e(edge_index_1[0]), pad_e(edge_index_1[1]))

    it = jnp.pad(jnp.transpose(input_, (1, 0, 2)),
                 ((0, 0), (0, 4), (0, 0))).reshape(160, D)
    stp = jnp.pad(state_, ((0, 4), (0, 0)))
    wp2 = jnp.pad(lin_prob_w, ((0, 7), (0, 0)))
    ws2 = jnp.pad(lin_sisr_w, ((0, 7), (0, 0)))
    bs2 = jnp.pad(lin_sisr_b[None, :], ((0, 7), (0, D - 1)))

    pcol, scol, hout = _final(
        e1out, it, stp, W_in.T.astype(f32),
        W_z[:, :D].T.astype(f32), W_z[:, D:].T.astype(f32),
        W_r[:, :D].T.astype(f32), W_r[:, D:].T.astype(f32),
        W_h[:, :D].T.astype(f32), W_h[:, D:].T.astype(f32),
        wp2, ws2, bs2)

    prob = pcol[:N_TOTAL, 0].reshape(B, N_NODES)[:, 1:]
    sisr = scol[:N_TOTAL, 0].reshape(B, N_NODES)
    h_out = hout[:B, :]
    return (prob, sisr, h_out)


# speedup vs baseline: 1.1370x; 1.1370x over previous
import functools

import jax
import jax.numpy as jnp
from jax import lax
from jax.experimental import pallas as pl
from jax.experimental.pallas import tpu as pltpu
from jax.experimental.pallas import tpu_sc as plsc

B = 4
N_NODES = 2500
N_TOTAL = B * N_NODES
N_PAD = 10240
E = 320000
D = 128
H = 64
NB = N_PAD // 512
NS = 16
EPS = 1e-16


def _dot(a, b):
    return jnp.dot(a, b, preferred_element_type=jnp.float32)


def _halves(c, a):
    return jnp.where(c == 0, a[:, :H], a[:, H:])


def _pre_body_common(c, g, wl_ref, bl_ref, wi_ref, ab_ref, wj_ref,
                     ajx_ref, init_ref):
    xl = _dot(g, wl_ref[...]) + bl_ref[0:1, :]
    aj = _dot(xl, wj_ref[...])
    ai = _dot(xl, wi_ref[...]) + ab_ref[0:1, :]
    xlh = _halves(c, xl)
    ajh = _halves(c, aj)
    aih = _halves(c, ai)
    s = aih + ajh
    le = jnp.exp(jnp.maximum(s, 0.2 * s))
    ajx_ref[...] = jnp.where(c == 2, ai, jnp.concatenate([ajh, xlh], axis=1))
    init_ref[...] = jnp.concatenate([le, xlh * le], axis=1)


def _pre0_kernel(g_ref, wl_ref, bl_ref, wi_ref, ab_ref, wj_ref,
                 ajx_ref, init_ref):
    c = pl.program_id(0)
    _pre_body_common(c, g_ref[...], wl_ref, bl_ref, wi_ref, ab_ref, wj_ref,
                     ajx_ref, init_ref)


def _pre1_kernel(e0_ref, e1_ref, wl_ref, bl_ref, wi_ref, ab_ref, wj_ref,
                 ajx_ref, init_ref):
    c = pl.program_id(0)
    e0 = e0_ref[...]
    e1 = e1_ref[...]
    g = jnp.concatenate([e0[:, H:] / (e0[:, :H] + EPS),
                         e1[:, H:] / (e1[:, :H] + EPS)], axis=1)
    _pre_body_common(c, g, wl_ref, bl_ref, wi_ref, ab_ref, wj_ref,
                     ajx_ref, init_ref)


def _pre_out_shapes():
    return [
        jax.ShapeDtypeStruct((3 * N_PAD, D), jnp.float32),
        jax.ShapeDtypeStruct((3 * N_PAD, D), jnp.float32),
    ]


_W_SPEC = pl.BlockSpec((D, D), lambda c, i: (0, 0))
_B_SPEC = pl.BlockSpec((8, D), lambda c, i: (0, 0))
_PRE_OUT_SPECS = [
    pl.BlockSpec((512, D), lambda c, i: (c * NB + i, 0)),
    pl.BlockSpec((512, D), lambda c, i: (c * NB + i, 0)),
]


def _pre0(g, wlT, bl2, wiT, ab2, wjT):
    return pl.pallas_call(
        _pre0_kernel,
        grid=(3, NB),
        in_specs=[pl.BlockSpec((512, D), lambda c, i: (i, 0)),
                  _W_SPEC, _B_SPEC, _W_SPEC, _B_SPEC, _W_SPEC],
        out_specs=_PRE_OUT_SPECS,
        out_shape=_pre_out_shapes(),
    )(g, wlT, bl2, wiT, ab2, wjT)


def _pre1(eout, wlT, bl2, wiT, ab2, wjT):
    return pl.pallas_call(
        _pre1_kernel,
        grid=(3, NB),
        in_specs=[pl.BlockSpec((512, D), lambda c, i: (i, 0)),
                  pl.BlockSpec((512, D), lambda c, i: (NB + i, 0)),
                  _W_SPEC, _B_SPEC, _W_SPEC, _B_SPEC, _W_SPEC],
        out_specs=_PRE_OUT_SPECS,
        out_shape=_pre_out_shapes(),
    )(eout, eout, wlT, bl2, wiT, ab2, wjT)


EP = 322560
EPW = EP // NS
CHUNK = 48
SUPER = 2016
NSUPER = EPW // SUPER
CPS = SUPER // CHUNK
ROWS_PW = N_PAD // NS
PAD_ROW = 10200


def _edge_body(tt_hbm, init_hbm, src_hbm, dst_hbm, out_hbm,
               src_all, dst_all,
               gi0, dsti0, gv0,
               gi1, dsti1, gv1,
               gi2, dsti2, gv2,
               acc, gsem0, gsem1, gsem2):
    c = lax.axis_index("c")
    sid = lax.axis_index("s")
    coff = (c * N_PAD).astype(jnp.int32)
    aoff = jnp.int32(2 * N_PAD)
    choff = c * H
    ebase = sid * EPW
    rbase = sid * ROWS_PW
    slots = ((gi0, dsti0, gv0, gsem0),
             (gi1, dsti1, gv1, gsem1),
             (gi2, dsti2, gv2, gsem2))

    pltpu.sync_copy(init_hbm.at[pl.ds(c * N_PAD + rbase, ROWS_PW)],
                    acc.at[pl.ds(rbase, ROWS_PW)])
    plsc.subcore_barrier()

    def prep(slot, t):
        gi, dsti, gv, gsem = slot
        tb = t * CHUNK
        for i in range(CHUNK // 16):
            sv = src_all[pl.ds(tb + i * 16, 16)]
            dv = dst_all[pl.ds(tb + i * 16, 16)]
            gi[pl.ds(i * 16, 16)] = sv + coff
            gi[pl.ds(CHUNK + i * 16, 16)] = dv + aoff
            dsti[pl.ds(i * 16, 16)] = dv
        pltpu.async_copy(tt_hbm.at[gi], gv, gsem)

    def consume(slot):
        gi, dsti, gv, gsem = slot
        pltpu.make_async_copy(tt_hbm.at[gi], gv, gsem).wait()

        @plsc.parallel_loop(0, CHUNK, unroll=4)
        def _(e):
            for k in range(H // 16):
                a = (gv[CHUNK + e, pl.ds(choff + k * 16, 16)]
                     + gv[e, pl.ds(k * 16, 16)])
                ex = jnp.exp(jnp.maximum(a, 0.2 * a))
                gv[e, pl.ds(k * 16, 16)] = ex
                gv[e, pl.ds(H + k * 16, 16)] = (
                    gv[e, pl.ds(H + k * 16, 16)] * ex)

        pltpu.sync_copy(gv.at[pl.ds(0, CHUNK)], acc.at[dsti], add=True)

    def super_body(u, _):
        sbase = ebase + u * SUPER
        pltpu.sync_copy(src_hbm.at[pl.ds(sbase, SUPER)], src_all)
        pltpu.sync_copy(dst_hbm.at[pl.ds(sbase, SUPER)], dst_all)
        for s in range(3):
            prep(slots[s], s)

        def round_body(tt, _):
            for s in range(3):
                consume(slots[s])
                prep(slots[s], 3 * tt + 3 + s)
            return 0

        lax.fori_loop(0, CPS // 3 - 1, round_body, 0)
        for s in range(3):
            consume(slots[s])
        return 0

    lax.fori_loop(0, NSUPER, super_body, 0)
    plsc.subcore_barrier()
    pltpu.sync_copy(acc.at[pl.ds(rbase, ROWS_PW)],
                    out_hbm.at[pl.ds(c * N_PAD + rbase, ROWS_PW)])


def _edge_pass(tt, init, src, dst):
    mesh = plsc.VectorSubcoreMesh(core_axis_name="c", subcore_axis_name="s",
                                  num_cores=2, num_subcores=NS)
    f = pl.kernel(
        _edge_body,
        out_type=jax.ShapeDtypeStruct((2 * N_PAD, D), jnp.float32),
        mesh=mesh,
        scratch_types=(
            [pltpu.VMEM((SUPER,), jnp.int32)] * 2
            + [pltpu.VMEM((2 * CHUNK,), jnp.int32),
               pltpu.VMEM((CHUNK,), jnp.int32),
               pltpu.VMEM((2 * CHUNK, D), jnp.float32)] * 3
            + [pltpu.VMEM_SHARED((N_PAD, D), jnp.float32),
               pltpu.SemaphoreType.DMA,
               pltpu.SemaphoreType.DMA,
               pltpu.SemaphoreType.DMA]
        ),
    )
    return f(tt, init, src, dst)


def _final_kernel(e0_ref, e1_ref, it_ref, st_ref, winT_ref,
                  wzsT_ref, wziT_ref, wrsT_ref, wriT_ref, whsT_ref, whiT_ref,
                  wp_ref, ws_ref, bs_ref,
                  pcol_ref, scol_ref, hout_ref):
    e0 = e0_ref[...]
    e1 = e1_ref[...]
    g1 = jnp.concatenate([e0[:, H:] / (e0[:, :H] + EPS),
                          e1[:, H:] / (e1[:, :H] + EPS)], axis=1)

    it = it_ref[...]
    m = jnp.zeros((8, D), jnp.float32)
    for l in range(20):
        m = m + it[l * 8:(l + 1) * 8, :]
    m = m * (1.0 / 20.0)
    inp = _dot(m, winT_ref[...])
    st = st_ref[...]
    z = jax.nn.sigmoid(_dot(st, wzsT_ref[...]) + _dot(inp, wziT_ref[...]))
    r = jax.nn.sigmoid(_dot(st, wrsT_ref[...]) + _dot(inp, wriT_ref[...]))
    hc = jnp.tanh(_dot(r * st, whsT_ref[...]) + _dot(inp, whiT_ref[...]))
    hh = (1.0 - z) * st + z * hc
    hout_ref[...] = hh

    n = lax.broadcasted_iota(jnp.int32, (N_PAD, 1), 0)
    b_id = n // N_NODES
    hh_exp = jnp.where(b_id == 0, hh[0:1, :],
             jnp.where(b_id == 1, hh[1:2, :],
             jnp.where(b_id == 2, hh[2:3, :], hh[3:4, :])))
    lp = jnp.sum(g1 * (hh_exp * wp_ref[0:1, :]), axis=1, keepdims=True)
    ls = jnp.sum(g1 * (hh_exp * ws_ref[0:1, :]), axis=1, keepdims=True)
    scol_ref[...] = jax.nn.sigmoid(ls + bs_ref[0:1, 0:1])

    valid = (n % N_NODES != 0) & (n < N_TOTAL)
    lpm = jnp.where(valid, lp, -1e30)
    mx = jnp.zeros((N_PAD, 1), jnp.float32)
    for b in range(B):
        inb = b_id == b
        mb = jnp.max(jnp.where(inb, lpm, -1e30))
        mx = jnp.where(inb, mb, mx)
    ex = jnp.where(valid, jnp.exp(lp - mx), 0.0)
    sm = jnp.zeros((N_PAD, 1), jnp.float32)
    for b in range(B):
        inb = b_id == b
        sb = jnp.sum(jnp.where(inb, ex, 0.0))
        sm = jnp.where(inb, sb, sm)
    pcol_ref[...] = ex / sm


def _final(eout1, it, st, winT, wzsT, wziT, wrsT, wriT, whsT, whiT,
           wp2, ws2, bs2):
    full = lambda r: pl.BlockSpec((r, D), lambda i: (0, 0))
    return pl.pallas_call(
        _final_kernel,
        grid=(1,),
        in_specs=[pl.BlockSpec((N_PAD, D), lambda i: (0, 0)),
                  pl.BlockSpec((N_PAD, D), lambda i: (1, 0)),
                  full(160), full(8), full(D),
                  full(D), full(D), full(D), full(D), full(D), full(D),
                  full(8), full(8), full(8)],
        out_specs=[pl.BlockSpec((N_PAD, 1), lambda i: (0, 0)),
                   pl.BlockSpec((N_PAD, 1), lambda i: (0, 0)),
                   pl.BlockSpec((8, D), lambda i: (0, 0))],
        out_shape=[jax.ShapeDtypeStruct((N_PAD, 1), jnp.float32),
                   jax.ShapeDtypeStruct((N_PAD, 1), jnp.float32),
                   jax.ShapeDtypeStruct((8, D), jnp.float32)],
    )(eout1, eout1, it, st, winT, wzsT, wziT, wrsT, wriT, whsT, whiT,
      wp2, ws2, bs2)


def kernel(x, edge_index_0, edge_index_1, state_, input_, W_in, W_z, W_r,
           W_h, gat0_lin_w, gat0_lin_b, gat0_attn_w, gat0_attn_b,
           gat1_lin_w, gat1_lin_b, gat1_attn_w, gat1_attn_b,
           lin_prob_w, lin_prob_b, lin_sisr_w, lin_sisr_b):
    f32 = jnp.float32
    xp = jnp.pad(x, ((0, N_PAD - N_TOTAL), (0, 0)))

    def pre_args(lin_w, lin_b, attn_w, attn_b):
        return (lin_w.T.astype(f32),
                jnp.pad(lin_b[None, :], ((0, 7), (0, 0))),
                attn_w[:, :D].T.astype(f32),
                jnp.pad(attn_b[None, :], ((0, 7), (0, 0))),
                attn_w[:, D:].T.astype(f32))

    def pad_e(v):
        return jnp.pad(v, (0, EP - E), constant_values=PAD_ROW)

    tt0, init0 = _pre0(xp, *pre_args(gat0_lin_w, gat0_lin_b,
                                     gat0_attn_w, gat0_attn_b))
    e0out = _edge_pass(tt0, init0,
                       pad_e(edge_index_0[0]), pad_e(edge_index_0[1]))
    tt1, init1 = _pre1(e0out, *pre_args(gat1_lin_w, gat1_lin_b,
                                        gat1_attn_w, gat1_attn_b))
    e1out = _edge_pass(tt1, init1,
                       pad_e(edge_index_1[0]), pad_e(edge_index_1[1]))

    it = jnp.pad(jnp.transpose(input_, (1, 0, 2)),
                 ((0, 0), (0, 4), (0, 0))).reshape(160, D)
    stp = jnp.pad(state_, ((0, 4), (0, 0)))
    wp2 = jnp.pad(lin_prob_w, ((0, 7), (0, 0)))
    ws2 = jnp.pad(lin_sisr_w, ((0, 7), (0, 0)))
    bs2 = jnp.pad(lin_sisr_b[None, :], ((0, 7), (0, D - 1)))

    pcol, scol, hout = _final(
        e1out, it, stp, W_in.T.astype(f32),
        W_z[:, :D].T.astype(f32), W_z[:, D:].T.astype(f32),
        W_r[:, :D].T.astype(f32), W_r[:, D:].T.astype(f32),
        W_h[:, :D].T.astype(f32), W_h[:, D:].T.astype(f32),
        wp2, ws2, bs2)

    prob = pcol[:N_TOTAL, 0].reshape(B, N_NODES)[:, 1:]
    sisr = scol[:N_TOTAL, 0].reshape(B, N_NODES)
    h_out = hout[:B, :]
    return (prob, sisr, h_out)
